# Initial kernel scaffold; baseline (speedup 1.0000x reference)
import jax
import jax.numpy as jnp
from jax.experimental import pallas as pl


def kernel(xyz):
    raise NotImplementedError("write your pallas kernel here")


# speedup vs baseline: 3.1893x; 3.1893x over previous
import functools

import jax
import jax.numpy as jnp
from jax import lax
from jax.experimental import pallas as pl
from jax.experimental.pallas import tpu as pltpu
from jax.experimental.pallas import tpu_sc as plsc

B = 8
N = 8192
G = 512
K = 32
GBLK = 8
NGB = G // GBLK
NC, NS, L = 2, 16, 16
NW = NC * NS
WPB = NW // B
GPW = G // WPB


def _fps_body(x_ref, y_ref, z_ref, ctr_ref, ssq_ref):
    X = x_ref[...]
    Y = y_ref[...]
    Z = z_ref[...]
    ssq_ref[...] = (X * X + Y * Y) + Z * Z
    lane = lax.broadcasted_iota(jnp.int32, (B, N), 1)
    glane = lax.broadcasted_iota(jnp.int32, (B, G), 1)

    def body(i, st):
        dist, fx, fy, fz, cxs, cys, czs = st
        gsel = glane == i
        cxs = jnp.where(gsel, fx, cxs)
        cys = jnp.where(gsel, fy, cys)
        czs = jnp.where(gsel, fz, czs)
        dxx = X - fx
        dyy = Y - fy
        dzz = Z - fz
        dn = (dxx * dxx + dyy * dyy) + dzz * dzz
        dist = jnp.minimum(dist, dn)
        m = jnp.max(dist, axis=1, keepdims=True)
        idxk = jnp.min(jnp.where(dist == m, lane, N), axis=1, keepdims=True)
        oh = lane == idxk
        fx = jnp.max(jnp.where(oh, X, -3e38), axis=1, keepdims=True)
        fy = jnp.max(jnp.where(oh, Y, -3e38), axis=1, keepdims=True)
        fz = jnp.max(jnp.where(oh, Z, -3e38), axis=1, keepdims=True)
        return (dist, fx, fy, fz, cxs, cys, czs)

    st0 = (
        jnp.full((B, N), 1e10, jnp.float32),
        X[:, 0:1],
        Y[:, 0:1],
        Z[:, 0:1],
        jnp.zeros((B, G), jnp.float32),
        jnp.zeros((B, G), jnp.float32),
        jnp.zeros((B, G), jnp.float32),
    )
    _, _, _, _, cxs, cys, czs = lax.fori_loop(0, G, body, st0)
    ctr_ref[0] = cxs
    ctr_ref[1] = cys
    ctr_ref[2] = czs


def _fps(x, y, z):
    return pl.pallas_call(
        _fps_body,
        out_shape=[
            jax.ShapeDtypeStruct((3, B, G), jnp.float32),
            jax.ShapeDtypeStruct((B, N), jnp.float32),
        ],
    )(x, y, z)


def _knn_body(ctr_ref, x_ref, y_ref, z_ref, ssq_ref, idx_ref):
    c = ctr_ref[0, 0]
    cx = c[:, 0:1]
    cy = c[:, 1:2]
    cz = c[:, 2:3]
    X = jnp.broadcast_to(x_ref[0], (GBLK, N))
    Y = jnp.broadcast_to(y_ref[0], (GBLK, N))
    Z = jnp.broadcast_to(z_ref[0], (GBLK, N))
    SX = jnp.broadcast_to(ssq_ref[0], (GBLK, N))
    sc = (cx * cx + cy * cy) + cz * cz
    cxb = cx.astype(jnp.bfloat16).astype(jnp.float32)
    cyb = cy.astype(jnp.bfloat16).astype(jnp.float32)
    czb = cz.astype(jnp.bfloat16).astype(jnp.float32)
    Xb = X.astype(jnp.bfloat16).astype(jnp.float32)
    Yb = Y.astype(jnp.bfloat16).astype(jnp.float32)
    Zb = Z.astype(jnp.bfloat16).astype(jnp.float32)
    dot = (cxb * Xb + cyb * Yb) + czb * Zb
    D = (sc - 2.0 * dot) + SX
    lane = lax.broadcasted_iota(jnp.int32, (GBLK, N), 1)
    klane = lax.broadcasted_iota(jnp.int32, (GBLK, K), 1)

    def body(k, st):
        D, acc = st
        m = jnp.min(D, axis=1, keepdims=True)
        idxk = jnp.min(jnp.where(D == m, lane, N), axis=1, keepdims=True)
        D = jnp.where(lane == idxk, 3.0e38, D)
        acc = jnp.where(klane == k, idxk, acc)
        return (D, acc)

    _, acc = lax.fori_loop(0, K, body, (D, jnp.zeros((GBLK, K), jnp.int32)))
    idx_ref[0] = acc


def _knn(ctr4, x, y, z, ssq):
    return pl.pallas_call(
        _knn_body,
        grid=(B, NGB),
        in_specs=[
            pl.BlockSpec((1, 1, GBLK, 3), lambda b, gb: (b, gb, 0, 0)),
            pl.BlockSpec((1, 1, N), lambda b, gb: (b, 0, 0)),
            pl.BlockSpec((1, 1, N), lambda b, gb: (b, 0, 0)),
            pl.BlockSpec((1, 1, N), lambda b, gb: (b, 0, 0)),
            pl.BlockSpec((1, 1, N), lambda b, gb: (b, 0, 0)),
        ],
        out_specs=pl.BlockSpec((1, GBLK, K), lambda b, gb: (b, gb, 0)),
        out_shape=jax.ShapeDtypeStruct((B, G, K), jnp.int32),
    )(ctr4, x.reshape(B, 1, N), y.reshape(B, 1, N),
      z.reshape(B, 1, N), ssq.reshape(B, 1, N))


def _sc_gather_body(x_hbm, y_hbm, z_hbm, idx_hbm, ctr_hbm, out_hbm,
                    x_v, y_v, z_v, idx_v, ctr_v, out_v):
    wid = lax.axis_index("s") * NC + lax.axis_index("c")
    b = wid // WPB
    w = wid % WPB
    pltpu.sync_copy(x_hbm.at[b], x_v)
    pltpu.sync_copy(y_hbm.at[b], y_v)
    pltpu.sync_copy(z_hbm.at[b], z_v)
    pltpu.sync_copy(idx_hbm.at[b, pl.ds(w * GPW * K, GPW * K)], idx_v)
    pltpu.sync_copy(ctr_hbm.at[b, pl.ds(w * GPW * 3, GPW * 3)], ctr_v)
    lanes = lax.iota(jnp.int32, L)

    def body(j, carry):
        j3 = jnp.full((L,), j * 3, jnp.int32)
        cxv = plsc.load_gather(ctr_v, [j3])
        cyv = plsc.load_gather(ctr_v, [j3 + 1])
        czv = plsc.load_gather(ctr_v, [j3 + 2])
        for t in range(K // L):
            iv = idx_v[pl.ds(j * K + t * L, L)]
            xg = plsc.load_gather(x_v, [iv])
            yg = plsc.load_gather(y_v, [iv])
            zg = plsc.load_gather(z_v, [iv])
            ov = (j * K + t * L + lanes) * 3
            plsc.store_scatter(out_v, [ov], xg - cxv)
            plsc.store_scatter(out_v, [ov + 1], yg - cyv)
            plsc.store_scatter(out_v, [ov + 2], zg - czv)
        return carry

    lax.fori_loop(0, GPW, body, 0)
    pltpu.sync_copy(out_v, out_hbm.at[b, pl.ds(w * GPW * K * 3, GPW * K * 3)])


@functools.cache
def _sc_gather():
    return pl.kernel(
        _sc_gather_body,
        out_type=jax.ShapeDtypeStruct((B, G * K * 3), jnp.float32),
        mesh=plsc.VectorSubcoreMesh(
            core_axis_name="c", subcore_axis_name="s",
            num_cores=NC, num_subcores=NS,
        ),
        compiler_params=pltpu.CompilerParams(needs_layout_passes=False),
        scratch_types=[
            pltpu.VMEM((N,), jnp.float32),
            pltpu.VMEM((N,), jnp.float32),
            pltpu.VMEM((N,), jnp.float32),
            pltpu.VMEM((GPW * K,), jnp.int32),
            pltpu.VMEM((GPW * 3,), jnp.float32),
            pltpu.VMEM((GPW * K * 3,), jnp.float32),
        ],
    )


def kernel(xyz):
    x = xyz[:, :, 0]
    y = xyz[:, :, 1]
    z = xyz[:, :, 2]
    ctr, ssq = _fps(x, y, z)
    center = jnp.transpose(ctr, (1, 2, 0))
    idx = _knn(center.reshape(B, NGB, GBLK, 3), x, y, z, ssq)
    neigh = _sc_gather()(x, y, z,
                       idx.reshape(B, G * K),
                       center.reshape(B, G * 3))
    return (neigh.reshape(B, G, K, 3), center)


# speedup vs baseline: 3.6011x; 1.1291x over previous
import functools

import jax
import jax.numpy as jnp
from jax import lax
from jax.experimental import pallas as pl
from jax.experimental.pallas import tpu as pltpu
from jax.experimental.pallas import tpu_sc as plsc

B = 8
N = 8192
G = 512
K = 32
NC, NS, L = 2, 16, 16
NW = NC * NS
WPB = NW // B
GPW = G // WPB
NCH = N // L
RBUF = 144
CB = RBUF + L
NCV = CB // L
IMAX = 0x7FFFFFFF


def _fps_body(x_ref, y_ref, z_ref, ctr_ref, ssq_ref):
    X = x_ref[...]
    Y = y_ref[...]
    Z = z_ref[...]
    ssq_ref[...] = (X * X + Y * Y) + Z * Z
    lane = lax.broadcasted_iota(jnp.int32, (B, N), 1)
    glane = lax.broadcasted_iota(jnp.int32, (B, G), 1)

    def body(i, st):
        dist, fx, fy, fz, cxs, cys, czs = st
        gsel = glane == i
        cxs = jnp.where(gsel, fx, cxs)
        cys = jnp.where(gsel, fy, cys)
        czs = jnp.where(gsel, fz, czs)
        dxx = X - fx
        dyy = Y - fy
        dzz = Z - fz
        dn = (dxx * dxx + dyy * dyy) + dzz * dzz
        dist = jnp.minimum(dist, dn)
        m = jnp.max(dist, axis=1, keepdims=True)
        idxk = jnp.min(jnp.where(dist == m, lane, N), axis=1, keepdims=True)
        oh = lane == idxk
        fx = jnp.max(jnp.where(oh, X, -3e38), axis=1, keepdims=True)
        fy = jnp.max(jnp.where(oh, Y, -3e38), axis=1, keepdims=True)
        fz = jnp.max(jnp.where(oh, Z, -3e38), axis=1, keepdims=True)
        return (dist, fx, fy, fz, cxs, cys, czs)

    st0 = (
        jnp.full((B, N), 1e10, jnp.float32),
        X[:, 0:1],
        Y[:, 0:1],
        Z[:, 0:1],
        jnp.zeros((B, G), jnp.float32),
        jnp.zeros((B, G), jnp.float32),
        jnp.zeros((B, G), jnp.float32),
    )
    _, _, _, _, cxs, cys, czs = lax.fori_loop(0, G, body, st0)
    ctr_ref[0] = cxs
    ctr_ref[1] = cys
    ctr_ref[2] = czs


def _fps(x, y, z):
    return pl.pallas_call(
        _fps_body,
        out_shape=[
            jax.ShapeDtypeStruct((3, B, G), jnp.float32),
            jax.ShapeDtypeStruct((B, N), jnp.float32),
        ],
    )(x, y, z)


def _scalar(v):
    return lax.reduce_max(v, (0,))


def _splat(s, dtype=jnp.int32):
    return jnp.full((L,), s, dtype)


def _rbf(v):
    u = plsc.bitcast(v, jnp.uint32)
    r = u + jnp.uint32(0x7FFF) + ((u >> jnp.uint32(16)) & jnp.uint32(1))
    return plsc.bitcast(r & jnp.uint32(0xFFFF0000), jnp.float32)


def _sc_knng_body(x_hbm, y_hbm, z_hbm, ssq_hbm,
                  ctr_hbm, out_hbm,
                  xb_v, yb_v, zb_v, x_v, y_v, z_v, ssq_v, ctr_v,
                  kbuf, h_v, cum_v, ckey, cidx, ibuf, out_v):
    wid = lax.axis_index("s") * NC + lax.axis_index("c")
    b = wid // WPB
    w = wid % WPB
    pltpu.sync_copy(x_hbm.at[b], x_v)
    pltpu.sync_copy(y_hbm.at[b], y_v)
    pltpu.sync_copy(z_hbm.at[b], z_v)
    pltpu.sync_copy(ssq_hbm.at[b], ssq_v)
    pltpu.sync_copy(ctr_hbm.at[b, pl.ds(w * GPW * 3, GPW * 3)], ctr_v)
    lanes = lax.iota(jnp.int32, L)
    zero16 = jnp.zeros((L,), jnp.int32)

    def round_points(c, _):
        sl = pl.ds(c * L, L)
        xb_v[sl] = _rbf(x_v[sl])
        yb_v[sl] = _rbf(y_v[sl])
        zb_v[sl] = _rbf(z_v[sl])
        return 0

    lax.fori_loop(0, NCH, round_points, 0)

    def find_bin(r):
        base = jnp.int32(0)
        cums = []
        for v in range(4):
            cv = plsc.cumsum(h_v[pl.ds(v * L, L)]) + base
            cums.append(cv)
            base = _scalar(cv)
        bb = None
        for v in range(4):
            fv = plsc.all_reduce_ffs(cums[v] >= r)
            cand = jnp.where(fv >= L, _splat(9999), fv + L * v)
            bb = cand if bb is None else jnp.minimum(bb, cand)
            cum_v[pl.ds(v * L, L)] = cums[v]
        bin_ = lax.reduce_min(bb, (0,))
        bs = _splat(bin_)
        cum_b = plsc.load_gather(cum_v, [bs])
        hist_b = plsc.load_gather(h_v, [bs])
        r_next = r - _scalar(cum_b - hist_b)
        return bin_, r_next

    def zero_hist():
        for v in range(4):
            h_v[pl.ds(v * L, L)] = zero16

    def center(j, _):
        j3 = _splat(j * 3)
        cx = plsc.load_gather(ctr_v, [j3])
        cy = plsc.load_gather(ctr_v, [j3 + 1])
        cz = plsc.load_gather(ctr_v, [j3 + 2])
        scq = (cx * cx + cy * cy) + cz * cz
        cxb = _rbf(cx)
        cyb = _rbf(cy)
        czb = _rbf(cz)

        zero_hist()

        def dbody(c, _):
            sl = pl.ds(c * L, L)
            dot = (cxb * xb_v[sl] + cyb * yb_v[sl]) + czb * zb_v[sl]
            d = (scq - 2.0 * dot) + ssq_v[sl]
            bits = plsc.bitcast(d, jnp.uint32)
            s = bits >> jnp.uint32(31)
            key = bits ^ (jnp.uint32(0x80000000) + s * jnp.uint32(0x7FFFFFFF))
            kbuf[sl] = key
            dig = (key >> jnp.uint32(26)).astype(jnp.int32)
            cnt, lastm = plsc.scan_count(dig)
            plsc.addupdate_scatter(h_v, [dig], cnt, mask=lastm)
            return 0

        lax.fori_loop(0, NCH, dbody, 0)
        b1, r2 = find_bin(jnp.int32(K))
        p1 = _splat(b1).astype(jnp.uint32)

        zero_hist()

        def h2body(c, _):
            key = kbuf[pl.ds(c * L, L)]
            elig = (key >> jnp.uint32(26)) == p1
            dig = ((key >> jnp.uint32(20)) & jnp.uint32(63)).astype(jnp.int32)
            cnt, lastm = plsc.scan_count(dig, mask=elig)
            plsc.addupdate_scatter(h_v, [dig], cnt, mask=lastm)
            return 0

        lax.fori_loop(0, NCH, h2body, 0)
        b2, r3 = find_bin(r2)
        p2 = (p1 << jnp.uint32(6)) | _splat(b2).astype(jnp.uint32)

        zero_hist()

        def h3body(c, _):
            key = kbuf[pl.ds(c * L, L)]
            elig = (key >> jnp.uint32(20)) == p2
            dig = ((key >> jnp.uint32(14)) & jnp.uint32(63)).astype(jnp.int32)
            cnt, lastm = plsc.scan_count(dig, mask=elig)
            plsc.addupdate_scatter(h_v, [dig], cnt, mask=lastm)
            return 0

        lax.fori_loop(0, NCH, h3body, 0)
        b3, _ = find_bin(r3)
        p3 = (p2 << jnp.uint32(6)) | _splat(b3).astype(jnp.uint32)
        thr = (p3 + jnp.uint32(1)) << jnp.uint32(14)

        for v in range(NCV):
            ckey[pl.ds(v * L, L)] = _splat(IMAX)
            cidx[pl.ds(v * L, L)] = _splat(IMAX)

        def cbody(c, cnt):
            key = kbuf[pl.ds(c * L, L)]
            m = key < thr
            skey = plsc.bitcast(key ^ jnp.uint32(0x80000000), jnp.int32)
            idxv = c * L + lanes
            plsc.store_compressed(ckey.at[pl.ds(cnt, L)], skey, mask=m)
            plsc.store_compressed(cidx.at[pl.ds(cnt, L)], idxv, mask=m)
            pc = plsc.all_reduce_population_count(m)
            return jnp.minimum(cnt + _scalar(pc), RBUF)

        lax.fori_loop(0, NCH, cbody, jnp.int32(0))

        def ebody(k, _):
            mn = None
            vs = []
            for v in range(NCV):
                kv = ckey[pl.ds(v * L, L)]
                vs.append(kv)
                mn = kv if mn is None else jnp.minimum(mn, kv)
            ms = _splat(lax.reduce_min(mn, (0,)))
            widv = None
            ivs = []
            for v in range(NCV):
                iv = cidx[pl.ds(v * L, L)]
                ivs.append(iv)
                cnd = jnp.where(vs[v] == ms, iv, _splat(IMAX))
                widv = cnd if widv is None else jnp.minimum(widv, cnd)
            ws = _splat(lax.reduce_min(widv, (0,)))
            plsc.store_scatter(ibuf, [_splat(k)], ws, mask=lanes == 0)
            for v in range(NCV):
                rm = (vs[v] == ms) & (ivs[v] == ws)
                ckey[pl.ds(v * L, L)] = jnp.where(rm, _splat(IMAX), vs[v])
            return 0

        lax.fori_loop(0, K, ebody, 0)

        for t in range(K // L):
            iv = ibuf[pl.ds(t * L, L)]
            xg = plsc.load_gather(x_v, [iv])
            yg = plsc.load_gather(y_v, [iv])
            zg = plsc.load_gather(z_v, [iv])
            ov = (j * K + t * L + lanes) * 3
            plsc.store_scatter(out_v, [ov], xg - cx)
            plsc.store_scatter(out_v, [ov + 1], yg - cy)
            plsc.store_scatter(out_v, [ov + 2], zg - cz)
        return 0

    lax.fori_loop(0, GPW, center, 0)
    pltpu.sync_copy(out_v, out_hbm.at[b, pl.ds(w * GPW * K * 3, GPW * K * 3)])


@functools.cache
def _sc_knng():
    return pl.kernel(
        _sc_knng_body,
        out_type=jax.ShapeDtypeStruct((B, G * K * 3), jnp.float32),
        mesh=plsc.VectorSubcoreMesh(
            core_axis_name="c", subcore_axis_name="s",
            num_cores=NC, num_subcores=NS,
        ),
        compiler_params=pltpu.CompilerParams(needs_layout_passes=False),
        scratch_types=[
            pltpu.VMEM((N,), jnp.float32),
            pltpu.VMEM((N,), jnp.float32),
            pltpu.VMEM((N,), jnp.float32),
            pltpu.VMEM((N,), jnp.float32),
            pltpu.VMEM((N,), jnp.float32),
            pltpu.VMEM((N,), jnp.float32),
            pltpu.VMEM((N,), jnp.float32),
            pltpu.VMEM((GPW * 3,), jnp.float32),
            pltpu.VMEM((N,), jnp.uint32),
            pltpu.VMEM((64,), jnp.int32),
            pltpu.VMEM((64,), jnp.int32),
            pltpu.VMEM((CB,), jnp.int32),
            pltpu.VMEM((CB,), jnp.int32),
            pltpu.VMEM((K,), jnp.int32),
            pltpu.VMEM((GPW * K * 3,), jnp.float32),
        ],
    )


def kernel(xyz):
    x = xyz[:, :, 0]
    y = xyz[:, :, 1]
    z = xyz[:, :, 2]
    ctr, ssq = _fps(x, y, z)
    center = jnp.transpose(ctr, (1, 2, 0))
    neigh = _sc_knng()(x, y, z, ssq, center.reshape(B, G * 3))
    return (neigh.reshape(B, G, K, 3), center)


# speedup vs baseline: 4.6779x; 1.2990x over previous
import functools

import jax
import jax.numpy as jnp
from jax import lax
from jax.experimental import pallas as pl
from jax.experimental.pallas import tpu as pltpu
from jax.experimental.pallas import tpu_sc as plsc

B = 8
N = 8192
G = 512
K = 32
NC, NS, L = 2, 16, 16
NW = NC * NS
WPB = NW // B
GPW = G // WPB
NCH = N // L
RBUF = 144
CB = RBUF + L
NCV = CB // L
IMAX = 0x7FFFFFFF


def _fps_body(x_ref, y_ref, z_ref, ctr_ref, ssq_ref):
    X = x_ref[...]
    Y = y_ref[...]
    Z = z_ref[...]
    ssq_ref[...] = (X * X + Y * Y) + Z * Z
    lane = lax.broadcasted_iota(jnp.int32, (B, N), 1)
    glane = lax.broadcasted_iota(jnp.int32, (B, G), 1)

    def body(i, st):
        dist, fx, fy, fz, cxs, cys, czs = st
        gsel = glane == i
        cxs = jnp.where(gsel, fx, cxs)
        cys = jnp.where(gsel, fy, cys)
        czs = jnp.where(gsel, fz, czs)
        dxx = X - fx
        dyy = Y - fy
        dzz = Z - fz
        dn = (dxx * dxx + dyy * dyy) + dzz * dzz
        dist = jnp.minimum(dist, dn)
        m = jnp.max(dist, axis=1, keepdims=True)
        idxk = jnp.min(jnp.where(dist == m, lane, N), axis=1, keepdims=True)
        oh = lane == idxk
        fx = jnp.max(jnp.where(oh, X, -3e38), axis=1, keepdims=True)
        fy = jnp.max(jnp.where(oh, Y, -3e38), axis=1, keepdims=True)
        fz = jnp.max(jnp.where(oh, Z, -3e38), axis=1, keepdims=True)
        return (dist, fx, fy, fz, cxs, cys, czs)

    st0 = (
        jnp.full((B, N), 1e10, jnp.float32),
        X[:, 0:1],
        Y[:, 0:1],
        Z[:, 0:1],
        jnp.zeros((B, G), jnp.float32),
        jnp.zeros((B, G), jnp.float32),
        jnp.zeros((B, G), jnp.float32),
    )
    _, _, _, _, cxs, cys, czs = lax.fori_loop(0, G, body, st0)
    ctr_ref[0] = cxs
    ctr_ref[1] = cys
    ctr_ref[2] = czs


def _fps(x, y, z):
    return pl.pallas_call(
        _fps_body,
        out_shape=[
            jax.ShapeDtypeStruct((3, B, G), jnp.float32),
            jax.ShapeDtypeStruct((B, N), jnp.float32),
        ],
    )(x, y, z)


def _scalar(v):
    return lax.reduce_max(v, (0,))


def _splat(s, dtype=jnp.int32):
    return jnp.full((L,), s, dtype)


def _rbf(v):
    u = plsc.bitcast(v, jnp.uint32)
    r = u + jnp.uint32(0x7FFF) + ((u >> jnp.uint32(16)) & jnp.uint32(1))
    return plsc.bitcast(r & jnp.uint32(0xFFFF0000), jnp.float32)


def _sc_knng_body(x_hbm, y_hbm, z_hbm, ssq_hbm,
                  ctr_hbm, out_hbm,
                  xb_v, yb_v, zb_v, x_v, y_v, z_v, ssq_v, ctr_v,
                  kbuf, h_v, cum_v, ckey, cidx, ibuf, out_v):
    wid = lax.axis_index("s") * NC + lax.axis_index("c")
    b = wid // WPB
    w = wid % WPB
    pltpu.sync_copy(x_hbm.at[b], x_v)
    pltpu.sync_copy(y_hbm.at[b], y_v)
    pltpu.sync_copy(z_hbm.at[b], z_v)
    pltpu.sync_copy(ssq_hbm.at[b], ssq_v)
    pltpu.sync_copy(ctr_hbm.at[b, pl.ds(w * GPW * 3, GPW * 3)], ctr_v)
    lanes = lax.iota(jnp.int32, L)
    zero16 = jnp.zeros((L,), jnp.int32)

    def round_points(c, _):
        sl = pl.ds(c * L, L)
        xb_v[sl] = _rbf(x_v[sl])
        yb_v[sl] = _rbf(y_v[sl])
        zb_v[sl] = _rbf(z_v[sl])
        return 0

    lax.fori_loop(0, NCH, round_points, 0)

    def find_bin(r):
        base = jnp.int32(0)
        cums = []
        for v in range(4):
            cv = plsc.cumsum(h_v[pl.ds(v * L, L)]) + base
            cums.append(cv)
            base = _scalar(cv)
        bb = None
        for v in range(4):
            fv = plsc.all_reduce_ffs(cums[v] >= r)
            cand = jnp.where(fv >= L, _splat(9999), fv + L * v)
            bb = cand if bb is None else jnp.minimum(bb, cand)
            cum_v[pl.ds(v * L, L)] = cums[v]
        bin_ = lax.reduce_min(bb, (0,))
        bs = _splat(bin_)
        cum_b = plsc.load_gather(cum_v, [bs])
        hist_b = plsc.load_gather(h_v, [bs])
        r_next = r - _scalar(cum_b - hist_b)
        return bin_, r_next

    def zero_hist():
        for v in range(4):
            h_v[pl.ds(v * L, L)] = zero16

    def center(j, _):
        j3 = _splat(j * 3)
        cx = plsc.load_gather(ctr_v, [j3])
        cy = plsc.load_gather(ctr_v, [j3 + 1])
        cz = plsc.load_gather(ctr_v, [j3 + 2])
        scq = (cx * cx + cy * cy) + cz * cz
        cxb = _rbf(cx)
        cyb = _rbf(cy)
        czb = _rbf(cz)

        zero_hist()

        def dbody(c4, _):
            for u in range(4):
                sl = pl.ds((c4 * 4 + u) * L, L)
                dot = (cxb * xb_v[sl] + cyb * yb_v[sl]) + czb * zb_v[sl]
                d = (scq - 2.0 * dot) + ssq_v[sl]
                bits = plsc.bitcast(d, jnp.uint32)
                s = bits >> jnp.uint32(31)
                key = bits ^ (jnp.uint32(0x80000000) + s * jnp.uint32(0x7FFFFFFF))
                kbuf[sl] = key
                dig = (key >> jnp.uint32(26)).astype(jnp.int32)
                cnt, lastm = plsc.scan_count(dig)
                plsc.addupdate_scatter(h_v, [dig], cnt, mask=lastm)
            return 0

        lax.fori_loop(0, NCH // 4, dbody, 0)
        b1, r2 = find_bin(jnp.int32(K))
        p1 = _splat(b1).astype(jnp.uint32)

        zero_hist()

        def h2body(c4, _):
            for u in range(4):
                key = kbuf[pl.ds((c4 * 4 + u) * L, L)]
                elig = (key >> jnp.uint32(26)) == p1
                dig = ((key >> jnp.uint32(20)) & jnp.uint32(63)).astype(jnp.int32)
                cnt, lastm = plsc.scan_count(dig, mask=elig)
                plsc.addupdate_scatter(h_v, [dig], cnt, mask=lastm)
            return 0

        lax.fori_loop(0, NCH // 4, h2body, 0)
        b2, _r3 = find_bin(r2)
        p2 = (p1 << jnp.uint32(6)) | _splat(b2).astype(jnp.uint32)
        thr = (p2 + jnp.uint32(1)) << jnp.uint32(20)

        for v in range(NCV):
            ckey[pl.ds(v * L, L)] = _splat(IMAX)
            cidx[pl.ds(v * L, L)] = _splat(IMAX)

        def cbody(c2, cnt):
            for u in range(2):
                c = c2 * 2 + u
                key = kbuf[pl.ds(c * L, L)]
                m = key < thr
                skey = plsc.bitcast(key ^ jnp.uint32(0x80000000), jnp.int32)
                idxv = c * L + lanes
                plsc.store_compressed(ckey.at[pl.ds(cnt, L)], skey, mask=m)
                plsc.store_compressed(cidx.at[pl.ds(cnt, L)], idxv, mask=m)
                pc = plsc.all_reduce_population_count(m)
                cnt = jnp.minimum(cnt + pc[0], RBUF)
            return cnt

        lax.fori_loop(0, NCH // 2, cbody, jnp.int32(0))

        def ebody(k, _):
            mn = None
            vs = []
            for v in range(NCV):
                kv = ckey[pl.ds(v * L, L)]
                vs.append(kv)
                mn = kv if mn is None else jnp.minimum(mn, kv)
            ms = _splat(lax.reduce_min(mn, (0,)))
            widv = None
            ivs = []
            for v in range(NCV):
                iv = cidx[pl.ds(v * L, L)]
                ivs.append(iv)
                cnd = jnp.where(vs[v] == ms, iv, _splat(IMAX))
                widv = cnd if widv is None else jnp.minimum(widv, cnd)
            ws = _splat(lax.reduce_min(widv, (0,)))
            plsc.store_scatter(ibuf, [_splat(k)], ws, mask=lanes == 0)
            for v in range(NCV):
                rm = (vs[v] == ms) & (ivs[v] == ws)
                ckey[pl.ds(v * L, L)] = jnp.where(rm, _splat(IMAX), vs[v])
            return 0

        lax.fori_loop(0, K, ebody, 0)

        for t in range(K // L):
            iv = ibuf[pl.ds(t * L, L)]
            xg = plsc.load_gather(x_v, [iv])
            yg = plsc.load_gather(y_v, [iv])
            zg = plsc.load_gather(z_v, [iv])
            ov = (j * K + t * L + lanes) * 3
            plsc.store_scatter(out_v, [ov], xg - cx)
            plsc.store_scatter(out_v, [ov + 1], yg - cy)
            plsc.store_scatter(out_v, [ov + 2], zg - cz)
        return 0

    lax.fori_loop(0, GPW, center, 0)
    pltpu.sync_copy(out_v, out_hbm.at[b, pl.ds(w * GPW * K * 3, GPW * K * 3)])


@functools.cache
def _sc_knng():
    return pl.kernel(
        _sc_knng_body,
        out_type=jax.ShapeDtypeStruct((B, G * K * 3), jnp.float32),
        mesh=plsc.VectorSubcoreMesh(
            core_axis_name="c", subcore_axis_name="s",
            num_cores=NC, num_subcores=NS,
        ),
        compiler_params=pltpu.CompilerParams(needs_layout_passes=False),
        scratch_types=[
            pltpu.VMEM((N,), jnp.float32),
            pltpu.VMEM((N,), jnp.float32),
            pltpu.VMEM((N,), jnp.float32),
            pltpu.VMEM((N,), jnp.float32),
            pltpu.VMEM((N,), jnp.float32),
            pltpu.VMEM((N,), jnp.float32),
            pltpu.VMEM((N,), jnp.float32),
            pltpu.VMEM((GPW * 3,), jnp.float32),
            pltpu.VMEM((N,), jnp.uint32),
            pltpu.VMEM((64,), jnp.int32),
            pltpu.VMEM((64,), jnp.int32),
            pltpu.VMEM((CB,), jnp.int32),
            pltpu.VMEM((CB,), jnp.int32),
            pltpu.VMEM((K,), jnp.int32),
            pltpu.VMEM((GPW * K * 3,), jnp.float32),
        ],
    )


def kernel(xyz):
    x = xyz[:, :, 0]
    y = xyz[:, :, 1]
    z = xyz[:, :, 2]
    ctr, ssq = _fps(x, y, z)
    center = jnp.transpose(ctr, (1, 2, 0))
    neigh = _sc_knng()(x, y, z, ssq, center.reshape(B, G * 3))
    return (neigh.reshape(B, G, K, 3), center)


# speedup vs baseline: 5.7505x; 1.2293x over previous
import functools

import jax
import jax.numpy as jnp
from jax import lax
from jax.experimental import pallas as pl
from jax.experimental.pallas import tpu as pltpu
from jax.experimental.pallas import tpu_sc as plsc

B = 8
N = 8192
G = 512
K = 32
NC, NS, L = 2, 16, 16
NW = NC * NS
WPB = NW // B
GPW = G // WPB
NCH = N // L
RBUF = 144
CB = RBUF + L
NCV = CB // L
IMAX = 0x7FFFFFFF


def _fps_body(x_ref, y_ref, z_ref, ctr_ref, ssq_ref):
    X = x_ref[...]
    Y = y_ref[...]
    Z = z_ref[...]
    ssq_ref[...] = (X * X + Y * Y) + Z * Z
    lane = lax.broadcasted_iota(jnp.int32, (B, N), 1)
    glane = lax.broadcasted_iota(jnp.int32, (B, G), 1)

    def body(i, st):
        dist, fx, fy, fz, cxs, cys, czs = st
        gsel = glane == i
        cxs = jnp.where(gsel, fx, cxs)
        cys = jnp.where(gsel, fy, cys)
        czs = jnp.where(gsel, fz, czs)
        dxx = X - fx
        dyy = Y - fy
        dzz = Z - fz
        dn = (dxx * dxx + dyy * dyy) + dzz * dzz
        dist = jnp.minimum(dist, dn)
        m = jnp.max(dist, axis=1, keepdims=True)
        idxk = jnp.min(jnp.where(dist == m, lane, N), axis=1, keepdims=True)
        oh = lane == idxk
        fx = jnp.max(jnp.where(oh, X, -3e38), axis=1, keepdims=True)
        fy = jnp.max(jnp.where(oh, Y, -3e38), axis=1, keepdims=True)
        fz = jnp.max(jnp.where(oh, Z, -3e38), axis=1, keepdims=True)
        return (dist, fx, fy, fz, cxs, cys, czs)

    st0 = (
        jnp.full((B, N), 1e10, jnp.float32),
        X[:, 0:1],
        Y[:, 0:1],
        Z[:, 0:1],
        jnp.zeros((B, G), jnp.float32),
        jnp.zeros((B, G), jnp.float32),
        jnp.zeros((B, G), jnp.float32),
    )
    _, _, _, _, cxs, cys, czs = lax.fori_loop(0, G, body, st0)
    ctr_ref[0] = cxs
    ctr_ref[1] = cys
    ctr_ref[2] = czs


def _fps(x, y, z):
    return pl.pallas_call(
        _fps_body,
        out_shape=[
            jax.ShapeDtypeStruct((3, B, G), jnp.float32),
            jax.ShapeDtypeStruct((B, N), jnp.float32),
        ],
    )(x, y, z)


def _scalar(v):
    return lax.reduce_max(v, (0,))


def _splat(s, dtype=jnp.int32):
    return jnp.full((L,), s, dtype)


def _rbf(v):
    u = plsc.bitcast(v, jnp.uint32)
    r = u + jnp.uint32(0x7FFF) + ((u >> jnp.uint32(16)) & jnp.uint32(1))
    return plsc.bitcast(r & jnp.uint32(0xFFFF0000), jnp.float32)


def _sc_knng_body(x_hbm, y_hbm, z_hbm, ssq_hbm,
                  ctr_hbm, out_hbm,
                  xb_v, yb_v, zb_v, x_v, y_v, z_v, ssq_v, ctr_v,
                  kbuf, kbuf2, h_v, cum_v, ckey, cidx, ibuf, out_v):
    wid = lax.axis_index("s") * NC + lax.axis_index("c")
    b = wid // WPB
    w = wid % WPB
    pltpu.sync_copy(x_hbm.at[b], x_v)
    pltpu.sync_copy(y_hbm.at[b], y_v)
    pltpu.sync_copy(z_hbm.at[b], z_v)
    pltpu.sync_copy(ssq_hbm.at[b], ssq_v)
    pltpu.sync_copy(ctr_hbm.at[b, pl.ds(w * GPW * 3, GPW * 3)], ctr_v)
    lanes = lax.iota(jnp.int32, L)
    zero16 = jnp.zeros((L,), jnp.int32)

    def round_points(c, _):
        sl = pl.ds(c * L, L)
        xb_v[sl] = _rbf(x_v[sl])
        yb_v[sl] = _rbf(y_v[sl])
        zb_v[sl] = _rbf(z_v[sl])
        return 0

    lax.fori_loop(0, NCH, round_points, 0)

    def find_bin(r, off):
        base = jnp.int32(0)
        cums = []
        for v in range(4):
            cv = plsc.cumsum(h_v[pl.ds(off + v * L, L)]) + base
            cums.append(cv)
            base = _scalar(cv)
        bb = None
        for v in range(4):
            fv = plsc.all_reduce_ffs(cums[v] >= r)
            cand = jnp.where(fv >= L, _splat(9999), fv + L * v)
            bb = cand if bb is None else jnp.minimum(bb, cand)
            cum_v[pl.ds(off + v * L, L)] = cums[v]
        bin_ = lax.reduce_min(bb, (0,))
        bs = _splat(off + bin_)
        cum_b = plsc.load_gather(cum_v, [bs])
        hist_b = plsc.load_gather(h_v, [bs])
        r_next = r - _scalar(cum_b - hist_b)
        return bin_, r_next

    def zero_hist():
        for v in range(8):
            h_v[pl.ds(v * L, L)] = zero16

    def one_center_coeffs(j):
        j3 = _splat(j * 3)
        cx = plsc.load_gather(ctr_v, [j3])
        cy = plsc.load_gather(ctr_v, [j3 + 1])
        cz = plsc.load_gather(ctr_v, [j3 + 2])
        scq = (cx * cx + cy * cy) + cz * cz
        return cx, cy, cz, scq, _rbf(cx), _rbf(cy), _rbf(cz)

    def center(p, _):
        jA = p * 2
        jB = p * 2 + 1
        cxA, cyA, czA, scqA, cxbA, cybA, czbA = one_center_coeffs(jA)
        cxB, cyB, czB, scqB, cxbB, cybB, czbB = one_center_coeffs(jB)

        zero_hist()

        def dbody(c2, _):
            for u in range(2):
                sl = pl.ds((c2 * 2 + u) * L, L)
                xv = xb_v[sl]
                yv = yb_v[sl]
                zv = zb_v[sl]
                sv = ssq_v[sl]
                for (cxb, cyb, czb, scq, kb, off) in (
                        (cxbA, cybA, czbA, scqA, kbuf, 0),
                        (cxbB, cybB, czbB, scqB, kbuf2, 64)):
                    dot = (cxb * xv + cyb * yv) + czb * zv
                    d = (scq - 2.0 * dot) + sv
                    bits = plsc.bitcast(d, jnp.uint32)
                    s = bits >> jnp.uint32(31)
                    key = bits ^ (jnp.uint32(0x80000000) + s * jnp.uint32(0x7FFFFFFF))
                    kb[sl] = key
                    dig = off + (key >> jnp.uint32(26)).astype(jnp.int32)
                    cnt, lastm = plsc.scan_count(dig)
                    plsc.addupdate_scatter(h_v, [dig], cnt, mask=lastm)
            return 0

        lax.fori_loop(0, NCH // 2, dbody, 0)
        b1A, r2A = find_bin(jnp.int32(K), 0)
        b1B, r2B = find_bin(jnp.int32(K), 64)
        p1A = _splat(b1A).astype(jnp.uint32)
        p1B = _splat(b1B).astype(jnp.uint32)

        zero_hist()

        def h2body(c2, _):
            for u in range(2):
                sl = pl.ds((c2 * 2 + u) * L, L)
                for (p1, kb, off) in ((p1A, kbuf, 0), (p1B, kbuf2, 64)):
                    key = kb[sl]
                    elig = (key >> jnp.uint32(26)) == p1
                    dig = off + ((key >> jnp.uint32(20)) & jnp.uint32(63)).astype(jnp.int32)
                    cnt, lastm = plsc.scan_count(dig, mask=elig)
                    plsc.addupdate_scatter(h_v, [dig], cnt, mask=lastm)
            return 0

        lax.fori_loop(0, NCH // 2, h2body, 0)
        b2A, _r3A = find_bin(r2A, 0)
        b2B, _r3B = find_bin(r2B, 64)
        thrA = (((p1A << jnp.uint32(6)) | _splat(b2A).astype(jnp.uint32))
                + jnp.uint32(1)) << jnp.uint32(20)
        thrB = (((p1B << jnp.uint32(6)) | _splat(b2B).astype(jnp.uint32))
                + jnp.uint32(1)) << jnp.uint32(20)

        def tail(j, kb, thr, cx, cy, cz):
            for v in range(NCV):
                ckey[pl.ds(v * L, L)] = _splat(IMAX)
                cidx[pl.ds(v * L, L)] = _splat(IMAX)

            def cbody(c2, cnt):
                for u in range(2):
                    c = c2 * 2 + u
                    key = kb[pl.ds(c * L, L)]
                    m = key < thr
                    skey = plsc.bitcast(key ^ jnp.uint32(0x80000000), jnp.int32)
                    idxv = c * L + lanes
                    plsc.store_compressed(ckey.at[pl.ds(cnt, L)], skey, mask=m)
                    plsc.store_compressed(cidx.at[pl.ds(cnt, L)], idxv, mask=m)
                    pc = plsc.all_reduce_population_count(m)
                    cnt = jnp.minimum(cnt + pc[0], RBUF)
                return cnt

            lax.fori_loop(0, NCH // 2, cbody, jnp.int32(0))

            def ebody(k, _):
                mn = None
                vs = []
                for v in range(NCV):
                    kv = ckey[pl.ds(v * L, L)]
                    vs.append(kv)
                    mn = kv if mn is None else jnp.minimum(mn, kv)
                ms = _splat(lax.reduce_min(mn, (0,)))
                widv = None
                ivs = []
                for v in range(NCV):
                    iv = cidx[pl.ds(v * L, L)]
                    ivs.append(iv)
                    cnd = jnp.where(vs[v] == ms, iv, _splat(IMAX))
                    widv = cnd if widv is None else jnp.minimum(widv, cnd)
                ws = _splat(lax.reduce_min(widv, (0,)))
                plsc.store_scatter(ibuf, [_splat(k)], ws, mask=lanes == 0)
                for v in range(NCV):
                    rm = (vs[v] == ms) & (ivs[v] == ws)
                    ckey[pl.ds(v * L, L)] = jnp.where(rm, _splat(IMAX), vs[v])
                return 0

            lax.fori_loop(0, K, ebody, 0)

            for t in range(K // L):
                iv = ibuf[pl.ds(t * L, L)]
                xg = plsc.load_gather(x_v, [iv])
                yg = plsc.load_gather(y_v, [iv])
                zg = plsc.load_gather(z_v, [iv])
                ov = (j * K + t * L + lanes) * 3
                plsc.store_scatter(out_v, [ov], xg - cx)
                plsc.store_scatter(out_v, [ov + 1], yg - cy)
                plsc.store_scatter(out_v, [ov + 2], zg - cz)

        tail(jA, kbuf, thrA, cxA, cyA, czA)
        tail(jB, kbuf2, thrB, cxB, cyB, czB)
        return 0

    lax.fori_loop(0, GPW // 2, center, 0)
    pltpu.sync_copy(out_v, out_hbm.at[b, pl.ds(w * GPW * K * 3, GPW * K * 3)])


@functools.cache
def _sc_knng():
    return pl.kernel(
        _sc_knng_body,
        out_type=jax.ShapeDtypeStruct((B, G * K * 3), jnp.float32),
        mesh=plsc.VectorSubcoreMesh(
            core_axis_name="c", subcore_axis_name="s",
            num_cores=NC, num_subcores=NS,
        ),
        compiler_params=pltpu.CompilerParams(needs_layout_passes=False),
        scratch_types=[
            pltpu.VMEM((N,), jnp.float32),
            pltpu.VMEM((N,), jnp.float32),
            pltpu.VMEM((N,), jnp.float32),
            pltpu.VMEM((N,), jnp.float32),
            pltpu.VMEM((N,), jnp.float32),
            pltpu.VMEM((N,), jnp.float32),
            pltpu.VMEM((N,), jnp.float32),
            pltpu.VMEM((GPW * 3,), jnp.float32),
            pltpu.VMEM((N,), jnp.uint32),
            pltpu.VMEM((N,), jnp.uint32),
            pltpu.VMEM((128,), jnp.int32),
            pltpu.VMEM((128,), jnp.int32),
            pltpu.VMEM((CB,), jnp.int32),
            pltpu.VMEM((CB,), jnp.int32),
            pltpu.VMEM((K,), jnp.int32),
            pltpu.VMEM((GPW * K * 3,), jnp.float32),
        ],
    )


def kernel(xyz):
    x = xyz[:, :, 0]
    y = xyz[:, :, 1]
    z = xyz[:, :, 2]
    ctr, ssq = _fps(x, y, z)
    center = jnp.transpose(ctr, (1, 2, 0))
    neigh = _sc_knng()(x, y, z, ssq, center.reshape(B, G * 3))
    return (neigh.reshape(B, G, K, 3), center)


# speedup vs baseline: 5.7606x; 1.0018x over previous
import functools

import jax
import jax.numpy as jnp
from jax import lax
from jax.experimental import pallas as pl
from jax.experimental.pallas import tpu as pltpu
from jax.experimental.pallas import tpu_sc as plsc

B = 8
N = 8192
G = 512
K = 32
NC, NS, L = 2, 16, 16
NW = NC * NS
WPB = NW // B
GPW = G // WPB
NCH = N // L
RBUF = 144
CB = RBUF + L
NCV = CB // L
IMAX = 0x7FFFFFFF


def _fps_body(x_ref, y_ref, z_ref, ctr_ref, ssq_ref):
    X = x_ref[...]
    Y = y_ref[...]
    Z = z_ref[...]
    ssq_ref[...] = (X * X + Y * Y) + Z * Z
    lane = lax.broadcasted_iota(jnp.int32, (B, N), 1)
    glane = lax.broadcasted_iota(jnp.int32, (B, G), 1)

    def body(i, st):
        dist, fx, fy, fz, cxs, cys, czs = st
        gsel = glane == i
        cxs = jnp.where(gsel, fx, cxs)
        cys = jnp.where(gsel, fy, cys)
        czs = jnp.where(gsel, fz, czs)
        dxx = X - fx
        dyy = Y - fy
        dzz = Z - fz
        dn = (dxx * dxx + dyy * dyy) + dzz * dzz
        dist = jnp.minimum(dist, dn)
        m = jnp.max(dist, axis=1, keepdims=True)
        idxk = jnp.min(jnp.where(dist == m, lane, N), axis=1, keepdims=True)
        oh = lane == idxk
        fx = jnp.max(jnp.where(oh, X, -3e38), axis=1, keepdims=True)
        fy = jnp.max(jnp.where(oh, Y, -3e38), axis=1, keepdims=True)
        fz = jnp.max(jnp.where(oh, Z, -3e38), axis=1, keepdims=True)
        return (dist, fx, fy, fz, cxs, cys, czs)

    st0 = (
        jnp.full((B, N), 1e10, jnp.float32),
        X[:, 0:1],
        Y[:, 0:1],
        Z[:, 0:1],
        jnp.zeros((B, G), jnp.float32),
        jnp.zeros((B, G), jnp.float32),
        jnp.zeros((B, G), jnp.float32),
    )
    _, _, _, _, cxs, cys, czs = lax.fori_loop(0, G, body, st0)
    ctr_ref[0] = cxs
    ctr_ref[1] = cys
    ctr_ref[2] = czs


def _fps(x, y, z):
    return pl.pallas_call(
        _fps_body,
        out_shape=[
            jax.ShapeDtypeStruct((3, B, G), jnp.float32),
            jax.ShapeDtypeStruct((B, N), jnp.float32),
        ],
    )(x, y, z)


def _scalar(v):
    return lax.reduce_max(v, (0,))


def _splat(s, dtype=jnp.int32):
    return jnp.full((L,), s, dtype)


def _rbf(v):
    u = plsc.bitcast(v, jnp.uint32)
    r = u + jnp.uint32(0x7FFF) + ((u >> jnp.uint32(16)) & jnp.uint32(1))
    return plsc.bitcast(r & jnp.uint32(0xFFFF0000), jnp.float32)


def _sc_knng_body(x_hbm, y_hbm, z_hbm, ssq_hbm,
                  ctr_hbm, out_hbm,
                  xb_v, yb_v, zb_v, x_v, y_v, z_v, ssq_v, ctr_v,
                  kbuf, kbuf2, h_v, cum_v, ckey, cidx, ibuf, out_v):
    wid = lax.axis_index("s") * NC + lax.axis_index("c")
    b = wid // WPB
    w = wid % WPB
    pltpu.sync_copy(x_hbm.at[b], x_v)
    pltpu.sync_copy(y_hbm.at[b], y_v)
    pltpu.sync_copy(z_hbm.at[b], z_v)
    pltpu.sync_copy(ssq_hbm.at[b], ssq_v)
    pltpu.sync_copy(ctr_hbm.at[b, pl.ds(w * GPW * 3, GPW * 3)], ctr_v)
    lanes = lax.iota(jnp.int32, L)
    zero16 = jnp.zeros((L,), jnp.int32)

    def round_points(c, _):
        sl = pl.ds(c * L, L)
        xb_v[sl] = _rbf(x_v[sl])
        yb_v[sl] = _rbf(y_v[sl])
        zb_v[sl] = _rbf(z_v[sl])
        return 0

    lax.fori_loop(0, NCH, round_points, 0)

    def find_bin(r, off):
        base = jnp.int32(0)
        cums = []
        for v in range(4):
            cv = plsc.cumsum(h_v[pl.ds(off + v * L, L)]) + base
            cums.append(cv)
            base = _scalar(cv)
        bb = None
        for v in range(4):
            fv = plsc.all_reduce_ffs(cums[v] >= r)
            cand = jnp.where(fv >= L, _splat(9999), fv + L * v)
            bb = cand if bb is None else jnp.minimum(bb, cand)
            cum_v[pl.ds(off + v * L, L)] = cums[v]
        bin_ = lax.reduce_min(bb, (0,))
        bs = _splat(off + bin_)
        cum_b = plsc.load_gather(cum_v, [bs])
        hist_b = plsc.load_gather(h_v, [bs])
        r_next = r - _scalar(cum_b - hist_b)
        return bin_, r_next

    def zero_hist():
        for v in range(8):
            h_v[pl.ds(v * L, L)] = zero16

    def one_center_coeffs(j):
        j3 = _splat(j * 3)
        cx = plsc.load_gather(ctr_v, [j3])
        cy = plsc.load_gather(ctr_v, [j3 + 1])
        cz = plsc.load_gather(ctr_v, [j3 + 2])
        scq = (cx * cx + cy * cy) + cz * cz
        return cx, cy, cz, scq, _rbf(cx), _rbf(cy), _rbf(cz)

    def center(p, _):
        jA = p * 2
        jB = p * 2 + 1
        cxA, cyA, czA, scqA, cxbA, cybA, czbA = one_center_coeffs(jA)
        cxB, cyB, czB, scqB, cxbB, cybB, czbB = one_center_coeffs(jB)

        zero_hist()

        def dbody(c2, _):
            for u in range(4):
                sl = pl.ds((c2 * 4 + u) * L, L)
                xv = xb_v[sl]
                yv = yb_v[sl]
                zv = zb_v[sl]
                sv = ssq_v[sl]
                for (cxb, cyb, czb, scq, kb, off) in (
                        (cxbA, cybA, czbA, scqA, kbuf, 0),
                        (cxbB, cybB, czbB, scqB, kbuf2, 64)):
                    dot = (cxb * xv + cyb * yv) + czb * zv
                    d = (scq - 2.0 * dot) + sv
                    bits = plsc.bitcast(d, jnp.uint32)
                    s = bits >> jnp.uint32(31)
                    key = bits ^ (jnp.uint32(0x80000000) + s * jnp.uint32(0x7FFFFFFF))
                    kb[sl] = key
                    dig = off + (key >> jnp.uint32(26)).astype(jnp.int32)
                    cnt, lastm = plsc.scan_count(dig)
                    plsc.addupdate_scatter(h_v, [dig], cnt, mask=lastm)
            return 0

        lax.fori_loop(0, NCH // 4, dbody, 0)
        b1A, r2A = find_bin(jnp.int32(K), 0)
        b1B, r2B = find_bin(jnp.int32(K), 64)
        p1A = _splat(b1A).astype(jnp.uint32)
        p1B = _splat(b1B).astype(jnp.uint32)

        zero_hist()

        def h2body(c2, _):
            for u in range(2):
                sl = pl.ds((c2 * 2 + u) * L, L)
                for (p1, kb, off) in ((p1A, kbuf, 0), (p1B, kbuf2, 64)):
                    key = kb[sl]
                    elig = (key >> jnp.uint32(26)) == p1
                    dig = off + ((key >> jnp.uint32(20)) & jnp.uint32(63)).astype(jnp.int32)
                    cnt, lastm = plsc.scan_count(dig, mask=elig)
                    plsc.addupdate_scatter(h_v, [dig], cnt, mask=lastm)
            return 0

        lax.fori_loop(0, NCH // 2, h2body, 0)
        b2A, _r3A = find_bin(r2A, 0)
        b2B, _r3B = find_bin(r2B, 64)
        thrA = (((p1A << jnp.uint32(6)) | _splat(b2A).astype(jnp.uint32))
                + jnp.uint32(1)) << jnp.uint32(20)
        thrB = (((p1B << jnp.uint32(6)) | _splat(b2B).astype(jnp.uint32))
                + jnp.uint32(1)) << jnp.uint32(20)

        def tail(j, kb, thr, cx, cy, cz):
            for v in range(NCV):
                ckey[pl.ds(v * L, L)] = _splat(IMAX)
                cidx[pl.ds(v * L, L)] = _splat(IMAX)

            def cbody(c2, cnt):
                for u in range(2):
                    c = c2 * 2 + u
                    key = kb[pl.ds(c * L, L)]
                    m = key < thr
                    skey = plsc.bitcast(key ^ jnp.uint32(0x80000000), jnp.int32)
                    idxv = c * L + lanes
                    plsc.store_compressed(ckey.at[pl.ds(cnt, L)], skey, mask=m)
                    plsc.store_compressed(cidx.at[pl.ds(cnt, L)], idxv, mask=m)
                    pc = plsc.all_reduce_population_count(m)
                    cnt = jnp.minimum(cnt + pc[0], RBUF)
                return cnt

            lax.fori_loop(0, NCH // 2, cbody, jnp.int32(0))

            def ebody(k, _):
                mn = None
                vs = []
                for v in range(NCV):
                    kv = ckey[pl.ds(v * L, L)]
                    vs.append(kv)
                    mn = kv if mn is None else jnp.minimum(mn, kv)
                ms = _splat(lax.reduce_min(mn, (0,)))
                widv = None
                ivs = []
                for v in range(NCV):
                    iv = cidx[pl.ds(v * L, L)]
                    ivs.append(iv)
                    cnd = jnp.where(vs[v] == ms, iv, _splat(IMAX))
                    widv = cnd if widv is None else jnp.minimum(widv, cnd)
                ws = _splat(lax.reduce_min(widv, (0,)))
                plsc.store_scatter(ibuf, [_splat(k)], ws, mask=lanes == 0)
                for v in range(NCV):
                    rm = (vs[v] == ms) & (ivs[v] == ws)
                    ckey[pl.ds(v * L, L)] = jnp.where(rm, _splat(IMAX), vs[v])
                return 0

            lax.fori_loop(0, K, ebody, 0)

            for t in range(K // L):
                iv = ibuf[pl.ds(t * L, L)]
                xg = plsc.load_gather(x_v, [iv])
                yg = plsc.load_gather(y_v, [iv])
                zg = plsc.load_gather(z_v, [iv])
                ov = (j * K + t * L + lanes) * 3
                plsc.store_scatter(out_v, [ov], xg - cx)
                plsc.store_scatter(out_v, [ov + 1], yg - cy)
                plsc.store_scatter(out_v, [ov + 2], zg - cz)

        tail(jA, kbuf, thrA, cxA, cyA, czA)
        tail(jB, kbuf2, thrB, cxB, cyB, czB)
        return 0

    lax.fori_loop(0, GPW // 2, center, 0)
    pltpu.sync_copy(out_v, out_hbm.at[b, pl.ds(w * GPW * K * 3, GPW * K * 3)])


@functools.cache
def _sc_knng():
    return pl.kernel(
        _sc_knng_body,
        out_type=jax.ShapeDtypeStruct((B, G * K * 3), jnp.float32),
        mesh=plsc.VectorSubcoreMesh(
            core_axis_name="c", subcore_axis_name="s",
            num_cores=NC, num_subcores=NS,
        ),
        compiler_params=pltpu.CompilerParams(needs_layout_passes=False),
        scratch_types=[
            pltpu.VMEM((N,), jnp.float32),
            pltpu.VMEM((N,), jnp.float32),
            pltpu.VMEM((N,), jnp.float32),
            pltpu.VMEM((N,), jnp.float32),
            pltpu.VMEM((N,), jnp.float32),
            pltpu.VMEM((N,), jnp.float32),
            pltpu.VMEM((N,), jnp.float32),
            pltpu.VMEM((GPW * 3,), jnp.float32),
            pltpu.VMEM((N,), jnp.uint32),
            pltpu.VMEM((N,), jnp.uint32),
            pltpu.VMEM((128,), jnp.int32),
            pltpu.VMEM((128,), jnp.int32),
            pltpu.VMEM((CB,), jnp.int32),
            pltpu.VMEM((CB,), jnp.int32),
            pltpu.VMEM((K,), jnp.int32),
            pltpu.VMEM((GPW * K * 3,), jnp.float32),
        ],
    )


def kernel(xyz):
    x = xyz[:, :, 0]
    y = xyz[:, :, 1]
    z = xyz[:, :, 2]
    ctr, ssq = _fps(x, y, z)
    center = jnp.transpose(ctr, (1, 2, 0))
    neigh = _sc_knng()(x, y, z, ssq, center.reshape(B, G * 3))
    return (neigh.reshape(B, G, K, 3), center)
